# Initial kernel scaffold; baseline (speedup 1.0000x reference)
import jax
import jax.numpy as jnp
from jax.experimental import pallas as pl


def kernel(x, emb_weight):
    raise NotImplementedError("write your pallas kernel here")


# speedup vs baseline: 8.5021x; 8.5021x over previous
import functools

import jax
import jax.numpy as jnp
from jax import lax
from jax.experimental import pallas as pl
from jax.experimental.pallas import tpu as pltpu
from jax.experimental.pallas import tpu_sc as plsc

D_MODEL = 128


def _normalize_table(w):
    v, d = w.shape
    scale = float(d) ** 0.5
    blk = 2000
    assert v % blk == 0

    def body(w_ref, o_ref):
        e = w_ref[...]
        n = jnp.sqrt(jnp.sum(e * e, axis=1, keepdims=True))
        o_ref[...] = e / jnp.maximum(n, 1e-12) * scale

    return pl.pallas_call(
        body,
        grid=(v // blk,),
        in_specs=[pl.BlockSpec((blk, d), lambda i: (i, 0))],
        out_specs=pl.BlockSpec((blk, d), lambda i: (i, 0)),
        out_shape=jax.ShapeDtypeStruct((v, d), jnp.float32),
    )(w)


@functools.lru_cache(maxsize=None)
def _make_sc_gather(n_idx, d):
    info = plsc.get_sparse_core_info()
    nc, ns = info.num_cores, info.num_subcores
    nw = nc * ns
    assert n_idx % nw == 0
    per_w = n_idx // nw
    chunk = 400
    assert per_w % chunk == 0 and chunk % 8 == 0
    n_chunks = per_w // chunk
    assert n_chunks % 2 == 0
    n_groups = n_chunks // 2

    mesh = plsc.VectorSubcoreMesh(core_axis_name="c", subcore_axis_name="s")

    @functools.partial(
        pl.kernel,
        mesh=mesh,
        out_type=jax.ShapeDtypeStruct((n_idx, d), jnp.float32),
        scratch_types=[
            pltpu.VMEM((chunk,), jnp.int32),
            pltpu.VMEM((chunk,), jnp.int32),
            pltpu.VMEM((chunk, d), jnp.float32),
            pltpu.VMEM((chunk, d), jnp.float32),
            pltpu.SemaphoreType.DMA,
            pltpu.SemaphoreType.DMA,
            pltpu.SemaphoreType.DMA,
        ],
    )
    def gather_kernel(
        table_hbm, idx_hbm, out_hbm, idx_v0, idx_v1, rows_v0, rows_v1,
        gsem, osem0, osem1,
    ):
        wid = lax.axis_index("s") * nc + lax.axis_index("c")
        base = wid * per_w
        idx_vs = (idx_v0, idx_v1)
        rows_vs = (rows_v0, rows_v1)
        osems = (osem0, osem1)

        def issue_gather(slot, i):
            off = base + i * chunk
            pltpu.sync_copy(idx_hbm.at[pl.ds(off, chunk)], idx_vs[slot])
            pltpu.async_copy(table_hbm.at[idx_vs[slot]], rows_vs[slot], gsem)

        def wait_gather(slot):
            pltpu.make_async_copy(
                table_hbm.at[idx_vs[slot]], rows_vs[slot], gsem
            ).wait()

        def issue_out(slot, i):
            off = base + i * chunk
            pltpu.async_copy(
                rows_vs[slot], out_hbm.at[pl.ds(off, chunk)], osems[slot]
            )

        def wait_out(slot):
            pltpu.make_async_copy(
                rows_vs[slot], out_hbm.at[pl.ds(base, chunk)], osems[slot]
            ).wait()

        issue_gather(0, 0)
        issue_gather(1, 1)
        wait_gather(0)
        issue_out(0, 0)
        wait_gather(1)
        issue_out(1, 1)

        def group(g, carry):
            i0 = 2 * g
            for slot in (0, 1):
                i = i0 + slot
                wait_out(slot)
                issue_gather(slot, i)
            for slot in (0, 1):
                wait_gather(slot)
                issue_out(slot, i0 + slot)
            return carry

        lax.fori_loop(1, n_groups, group, 0)
        wait_out(0)
        wait_out(1)

    return gather_kernel


def kernel(x, emb_weight):
    b, t = x.shape
    v, d = emb_weight.shape
    table = _normalize_table(emb_weight)
    idx = x.reshape(-1).astype(jnp.int32)
    out = _make_sc_gather(b * t, d)(table, idx)
    return out.reshape(b, t, d)
